# baseline (speedup 1.0000x reference)
import jax
import jax.numpy as jnp
from jax.experimental import pallas as pl
from jax.experimental.pallas import tpu as pltpu

_BATCH = 16384
_BLOCK = 1024


def _mlp_kernel(x_ref, w1_ref, b1_ref, m1_ref, w2_ref, b2_ref, m2_ref,
                w3_ref, b3_ref, m3_ref, w4_ref, b4_ref, m4_ref,
                w5_ref, b5_ref, m5_ref,
                h1_ref, h2_ref, h3_ref, h4_ref, h5_ref):
    dn = (((1,), (1,)), ((), ()))

    x = x_ref[...]
    w1 = w1_ref[...] * m1_ref[...]
    h1 = jax.lax.dot_general(x, w1, dn, preferred_element_type=jnp.float32)
    h1 = jnp.maximum(h1 + b1_ref[...], 0.0)
    h1_ref[...] = h1

    w2 = w2_ref[...] * m2_ref[...]
    h2 = jax.lax.dot_general(h1, w2, dn, preferred_element_type=jnp.float32)
    h2 = jnp.maximum(h2 + b2_ref[...], 0.0)
    h2_ref[...] = h2

    w3 = w3_ref[...] * m3_ref[...]
    h3 = jax.lax.dot_general(h2, w3, dn, preferred_element_type=jnp.float32)
    h3 = jnp.maximum(h3 + b3_ref[...], 0.0)
    h3_ref[...] = h3

    w4 = w4_ref[...] * m4_ref[...]
    h4 = jax.lax.dot_general(h3, w4, dn, preferred_element_type=jnp.float32)
    h4 = h4 + b4_ref[...]
    h4_ref[...] = h4

    w5 = w5_ref[...] * m5_ref[...]
    h5 = jax.lax.dot_general(h4, w5, dn, preferred_element_type=jnp.float32)
    h5 = h5 + b5_ref[...]
    h5_ref[...] = h5


def _fused_mlp(x, W1, b1, M1, W2, b2, M2, W3, b3, M3, W4, b4, M4, W5, b5, M5,
               block):
    n = x.shape[0]
    d_in = x.shape[1]
    d1, d2, d3, d4, d5 = W1.shape[0], W2.shape[0], W3.shape[0], W4.shape[0], W5.shape[0]
    b1, b2, b3, b4, b5 = (b.reshape(1, -1) for b in (b1, b2, b3, b4, b5))

    def wspec(w):
        return pl.BlockSpec(w.shape, lambda i: (0, 0))

    grid = (n // block,)
    in_specs = [
        pl.BlockSpec((block, d_in), lambda i: (i, 0)),
        wspec(W1), wspec(b1), wspec(M1),
        wspec(W2), wspec(b2), wspec(M2),
        wspec(W3), wspec(b3), wspec(M3),
        wspec(W4), wspec(b4), wspec(M4),
        wspec(W5), wspec(b5), wspec(M5),
    ]
    out_specs = [
        pl.BlockSpec((block, d1), lambda i: (i, 0)),
        pl.BlockSpec((block, d2), lambda i: (i, 0)),
        pl.BlockSpec((block, d3), lambda i: (i, 0)),
        pl.BlockSpec((block, d4), lambda i: (i, 0)),
        pl.BlockSpec((block, d5), lambda i: (i, 0)),
    ]
    out_shapes = [
        jax.ShapeDtypeStruct((n, d1), jnp.float32),
        jax.ShapeDtypeStruct((n, d2), jnp.float32),
        jax.ShapeDtypeStruct((n, d3), jnp.float32),
        jax.ShapeDtypeStruct((n, d4), jnp.float32),
        jax.ShapeDtypeStruct((n, d5), jnp.float32),
    ]
    return pl.pallas_call(
        _mlp_kernel,
        grid=grid,
        in_specs=in_specs,
        out_specs=out_specs,
        out_shape=out_shapes,
        compiler_params=pltpu.CompilerParams(
            dimension_semantics=("arbitrary",),
        ),
    )(x, W1, b1, M1, W2, b2, M2, W3, b3, M3, W4, b4, M4, W5, b5, M5)


def kernel(x, W1, b1, M1, W2, b2, M2, W3, b3, M3, W4, b4, M4, W5, b5, M5):
    h1, h2, h3, h4, h5 = _fused_mlp(
        x, W1, b1, M1, W2, b2, M2, W3, b3, M3, W4, b4, M4, W5, b5, M5,
        _BLOCK)
    return (h5, h1, h2, h3, h4, h5)
